# baseline (speedup 1.0000x reference)
import functools

import jax
import jax.numpy as jnp
from jax import lax
from jax.experimental import pallas as pl
from jax.experimental.pallas import tpu as pltpu
from jax.experimental.pallas import tpu_sc as plsc

B = 32
NN = 1000
N = B * NN
E = 512000
D_IN = 128
H1 = 256
H2 = 128
ENC = 256
DEC = 128
TEMB = 200

NC = 2
NS = 16
LANES = 16
ROUNDS = 2
QTR = N // (NC * ROUNDS)
ACC_ROWS = 8192

F = 128
SEG_CHUNK = 128
DEC_CHUNK = 80


def _sc_mesh():
    return plsc.VectorSubcoreMesh(core_axis_name="c", subcore_axis_name="s",
                                  num_cores=NC, num_subcores=NS)


_DNUMS = lax.GatherDimensionNumbers(
    offset_dims=(), collapsed_slice_dims=(0,), start_index_map=(0,))


def _shuf(v, p):
    return lax.gather(v, p[:, None], _DNUMS, slice_sizes=(1,),
                      mode=lax.GatherScatterMode.PROMISE_IN_BOUNDS)


SEG_STAGE = 2000
FIRE = 128
PEND = 160


def _segsum_body(vals_hbm, src_hbm, dst_hbm, ea_hbm, zeros_hbm, out_hbm,
                 acc_sh, st_src, st_dst, st_ea, p_src, p_dst, p_ea,
                 fire_src, fire_dst, rows_v, sem):
    c = lax.axis_index("c")
    s = lax.axis_index("s")
    ept = E // NS
    rpt = ACC_ROWS // NS
    zero16i = jnp.zeros((LANES,), jnp.int32)
    zero16f = jnp.zeros((LANES,), jnp.float32)
    lanes = lax.iota(jnp.int32, LANES)
    for g in range(PEND // LANES):
        p_src[pl.ds(g * LANES, LANES)] = zero16i
        p_dst[pl.ds(g * LANES, LANES)] = zero16i
        p_ea[pl.ds(g * LANES, LANES)] = zero16f

    def fire():
        for g in range(FIRE // LANES):
            sl = pl.ds(g * LANES, LANES)
            fire_src[sl] = p_src[sl]
            fire_dst[sl] = p_dst[sl]
        pltpu.async_copy(vals_hbm.at[fire_src], rows_v, sem).wait()

        def scale(k, _):
            ea16 = p_ea[pl.ds(k * LANES, LANES)]
            for e in range(LANES):
                m = jnp.full((LANES,), ea16[e], jnp.float32)
                row = k * LANES + e
                for j in range(F // LANES):
                    sl = pl.ds(j * LANES, LANES)
                    rows_v[row, sl] = rows_v[row, sl] * m
            return 0

        lax.fori_loop(0, FIRE // LANES, scale, 0)
        pltpu.sync_copy(rows_v, acc_sh.at[fire_dst], add=True)

    for r in range(ROUNDS):
        base_node = (r * NC + c) * QTR
        pltpu.sync_copy(zeros_hbm.at[pl.ds(s * rpt, rpt)],
                        acc_sh.at[pl.ds(s * rpt, rpt)])
        plsc.subcore_barrier()

        def stage(t, cur):
            ebase = s * ept + t * SEG_STAGE
            pltpu.sync_copy(src_hbm.at[pl.ds(ebase, SEG_STAGE)], st_src)
            pltpu.sync_copy(dst_hbm.at[pl.ds(ebase, SEG_STAGE)], st_dst)
            pltpu.sync_copy(ea_hbm.at[pl.ds(ebase, SEG_STAGE)], st_ea)

            def grp(g, cur):
                sl = pl.ds(g * LANES, LANES)
                local = st_dst[sl] - base_node
                m = (local >= 0) & (local < QTR)
                m_f = jnp.where(m, 1.0, 0.0)
                incl = m_f
                for sh in (1, 2, 4, 8):
                    t = _shuf(incl, jnp.maximum(lanes - sh, 0))
                    incl = incl + jnp.where(lanes >= sh, t, 0.0)
                excl = (incl - m_f).astype(jnp.int32)
                pos = jnp.where(m, cur + excl, PEND + lanes)
                plsc.store_scatter(p_dst, [pos], local)
                plsc.store_scatter(p_src, [pos], st_src[sl])
                plsc.store_scatter(p_ea, [pos], st_ea[sl])
                cur = cur + incl[LANES - 1].astype(jnp.int32)

                @pl.when(cur >= FIRE)
                def _():
                    fire()
                    for a in (p_src, p_dst, p_ea):
                        a[pl.ds(0, LANES)] = a[pl.ds(FIRE, LANES)]

                return jnp.where(cur >= FIRE, cur - FIRE, cur)

            return lax.fori_loop(0, SEG_STAGE // LANES, grp, cur)

        cur = lax.fori_loop(0, ept // SEG_STAGE, stage, jnp.int32(0))
        for g in range(FIRE // LANES):
            sl = pl.ds(g * LANES, LANES)
            keep = (lanes + g * LANES) < cur
            p_ea[sl] = jnp.where(keep, p_ea[sl], 0.0)
        fire()
        plsc.subcore_barrier()
        @pl.when(s < 10)
        def _():
            pltpu.sync_copy(acc_sh.at[pl.ds(s * 800, 800)],
                            out_hbm.at[pl.ds(base_node + s * 800, 800)])
        plsc.subcore_barrier()


def _segsum(vals, src, dst, ea, zeros):
    k = pl.kernel(
        _segsum_body,
        out_type=jax.ShapeDtypeStruct((N, F), jnp.float32),
        mesh=_sc_mesh(),
        compiler_params=pltpu.CompilerParams(needs_layout_passes=False),
        scratch_types=[
            pltpu.VMEM_SHARED((ACC_ROWS, F), jnp.float32),
            pltpu.VMEM((SEG_STAGE,), jnp.int32),
            pltpu.VMEM((SEG_STAGE,), jnp.int32),
            pltpu.VMEM((SEG_STAGE,), jnp.float32),
            pltpu.VMEM((PEND + LANES,), jnp.int32),
            pltpu.VMEM((PEND + LANES,), jnp.int32),
            pltpu.VMEM((PEND + LANES,), jnp.float32),
            pltpu.VMEM((FIRE,), jnp.int32),
            pltpu.VMEM((FIRE,), jnp.int32),
            pltpu.VMEM((FIRE, F), jnp.float32),
            pltpu.SemaphoreType.DMA,
        ],
    )
    return k(vals, src, dst, ea, zeros)


def _edgedec_body(a_hbm, b_hbm, src_hbm, dst_hbm, w2_hbm, out_hbm,
                  idx_s0, idx_d0, idx_s1, idx_d1, ar0, br0, ar1, br1,
                  w2_v, outc, acc2d, sem0, sem1):
    c = lax.axis_index("c")
    s = lax.axis_index("s")
    wid = c * NS + s
    ept = E // (NC * NS)
    n_chunks = ept // DEC_CHUNK
    pltpu.sync_copy(w2_hbm, w2_v)
    b2v = w2_v[pl.ds(H1, LANES)]
    w2sl = [w2_v[pl.ds(j * LANES, LANES)] for j in range(H1 // LANES)]
    lanes = lax.iota(jnp.int32, LANES)

    def issue(i, idx_s, idx_d, ar, br, sem):
        ebase = wid * ept + i * DEC_CHUNK
        pltpu.sync_copy(src_hbm.at[pl.ds(ebase, DEC_CHUNK)], idx_s)
        pltpu.sync_copy(dst_hbm.at[pl.ds(ebase, DEC_CHUNK)], idx_d)
        da = pltpu.async_copy(a_hbm.at[idx_s], ar, sem)
        db = pltpu.async_copy(b_hbm.at[idx_d], br, sem)
        return da, db

    def compute(i, arows, brows):
        def grp(g, _):
            for e in range(LANES):
                row = g * LANES + e
                acc0 = jnp.zeros((LANES,), jnp.float32)
                acc1 = jnp.zeros((LANES,), jnp.float32)
                for j in range(0, H1 // LANES, 2):
                    sl0 = pl.ds(j * LANES, LANES)
                    sl1 = pl.ds((j + 1) * LANES, LANES)
                    t0 = jnp.maximum(arows[row, sl0] + brows[row, sl0], 0.0)
                    t1 = jnp.maximum(arows[row, sl1] + brows[row, sl1], 0.0)
                    acc0 = acc0 + t0 * w2sl[j]
                    acc1 = acc1 + t1 * w2sl[j + 1]
                acc2d[e, :] = acc0 + acc1
            cols = [plsc.load_gather(
                acc2d, [lanes, jnp.full((LANES,), j, jnp.int32)])
                for j in range(LANES)]
            while len(cols) > 1:
                cols = [cols[k] + cols[k + 1] for k in range(0, len(cols), 2)]
            t = cols[0] + b2v
            outc[pl.ds(g * LANES, LANES)] = 1.0 / (1.0 + jnp.exp(-t))
            return 0

        lax.fori_loop(0, DEC_CHUNK // LANES, grp, 0)
        ebase = wid * ept + i * DEC_CHUNK
        pltpu.sync_copy(outc, out_hbm.at[pl.ds(ebase, DEC_CHUNK)])

    d0 = issue(0, idx_s0, idx_d0, ar0, br0, sem0)

    def pipe(h, _):
        i0 = 2 * h
        d1 = issue(i0 + 1, idx_s1, idx_d1, ar1, br1, sem1)
        for d in d0:
            d.wait()
        compute(i0, ar0, br0)

        @pl.when(h < n_chunks // 2 - 1)
        def _():
            issue(i0 + 2, idx_s0, idx_d0, ar0, br0, sem0)

        for d in d1:
            d.wait()
        compute(i0 + 1, ar1, br1)
        return 0

    lax.fori_loop(0, n_chunks // 2, pipe, 0)


def _edgedec(A, Bmb, src, dst, w2ext):
    k = pl.kernel(
        _edgedec_body,
        out_type=jax.ShapeDtypeStruct((E,), jnp.float32),
        mesh=_sc_mesh(),
        compiler_params=pltpu.CompilerParams(needs_layout_passes=False),
        scratch_types=[
            pltpu.VMEM((DEC_CHUNK,), jnp.int32),
            pltpu.VMEM((DEC_CHUNK,), jnp.int32),
            pltpu.VMEM((DEC_CHUNK,), jnp.int32),
            pltpu.VMEM((DEC_CHUNK,), jnp.int32),
            pltpu.VMEM((DEC_CHUNK, H1), jnp.float32),
            pltpu.VMEM((DEC_CHUNK, H1), jnp.float32),
            pltpu.VMEM((DEC_CHUNK, H1), jnp.float32),
            pltpu.VMEM((DEC_CHUNK, H1), jnp.float32),
            pltpu.VMEM((H1 + LANES,), jnp.float32),
            pltpu.VMEM((DEC_CHUNK,), jnp.float32),
            pltpu.VMEM((LANES, LANES), jnp.float32),
            pltpu.SemaphoreType.DMA,
            pltpu.SemaphoreType.DMA,
        ],
    )
    return k(A, Bmb, src, dst, w2ext)


_RB = 800


def _lin1_tc(agg_ref, x_ref, wr_ref, wq_ref, b_ref, o_ref):
    o_ref[...] = jnp.maximum(
        jnp.dot(agg_ref[...], wr_ref[...], preferred_element_type=jnp.float32, precision=lax.Precision.HIGHEST)
        + jnp.dot(x_ref[...], wq_ref[...], preferred_element_type=jnp.float32, precision=lax.Precision.HIGHEST)
        + b_ref[...], 0.0)


def _lin1(agg, x, wrT, wqT, b):
    return pl.pallas_call(
        _lin1_tc,
        grid=(N // _RB,),
        in_specs=[
            pl.BlockSpec((_RB, D_IN), lambda i: (i, 0)),
            pl.BlockSpec((_RB, D_IN), lambda i: (i, 0)),
            pl.BlockSpec((D_IN, H1), lambda i: (0, 0)),
            pl.BlockSpec((D_IN, H1), lambda i: (0, 0)),
            pl.BlockSpec((1, H1), lambda i: (0, 0)),
        ],
        out_specs=pl.BlockSpec((_RB, H1), lambda i: (i, 0)),
        out_shape=jax.ShapeDtypeStruct((N, H1), jnp.float32),
    )(agg, x, wrT, wqT, b)


def _lin2pre_tc(h_ref, wr_ref, wq_ref, b_ref, y_ref, r_ref):
    h = h_ref[...]
    y_ref[...] = jnp.dot(h, wr_ref[...], preferred_element_type=jnp.float32, precision=lax.Precision.HIGHEST)
    r_ref[...] = (jnp.dot(h, wq_ref[...], preferred_element_type=jnp.float32, precision=lax.Precision.HIGHEST)
                  + b_ref[...])


def _lin2pre(h1, wrT, wqT, b):
    return pl.pallas_call(
        _lin2pre_tc,
        grid=(N // _RB,),
        in_specs=[
            pl.BlockSpec((_RB, H1), lambda i: (i, 0)),
            pl.BlockSpec((H1, H2), lambda i: (0, 0)),
            pl.BlockSpec((H1, H2), lambda i: (0, 0)),
            pl.BlockSpec((1, H2), lambda i: (0, 0)),
        ],
        out_specs=[
            pl.BlockSpec((_RB, H2), lambda i: (i, 0)),
            pl.BlockSpec((_RB, H2), lambda i: (i, 0)),
        ],
        out_shape=[
            jax.ShapeDtypeStruct((N, H2), jnp.float32),
            jax.ShapeDtypeStruct((N, H2), jnp.float32),
        ],
    )(h1, wrT, wqT, b)


def _h2ew_tc(agg_ref, r_ref, o_ref):
    o_ref[...] = jnp.maximum(agg_ref[...] + r_ref[...], 0.0)


def _h2ew(agg2, r2):
    return pl.pallas_call(
        _h2ew_tc,
        grid=(N // _RB,),
        in_specs=[
            pl.BlockSpec((_RB, H2), lambda i: (i, 0)),
            pl.BlockSpec((_RB, H2), lambda i: (i, 0)),
        ],
        out_specs=pl.BlockSpec((_RB, H2), lambda i: (i, 0)),
        out_shape=jax.ShapeDtypeStruct((N, H2), jnp.float32),
    )(agg2, r2)


def _emb_tc(hour_ref, week_ref, ht_ref, wt_ref, eh_ref, ew_ref):
    oh = (hour_ref[...] == lax.broadcasted_iota(jnp.int32, (B, 24), 1)
          ).astype(jnp.float32)
    ow = (week_ref[...] == lax.broadcasted_iota(jnp.int32, (B, 7), 1)
          ).astype(jnp.float32)
    eh_ref[...] = jnp.dot(oh, ht_ref[...], preferred_element_type=jnp.float32, precision=lax.Precision.HIGHEST)
    ew_ref[...] = jnp.dot(ow, wt_ref[...], preferred_element_type=jnp.float32, precision=lax.Precision.HIGHEST)


def _emb(hour2, week2, hour_table, week_table):
    return pl.pallas_call(
        _emb_tc,
        out_shape=[
            jax.ShapeDtypeStruct((B, TEMB), jnp.float32),
            jax.ShapeDtypeStruct((B, TEMB), jnp.float32),
        ],
    )(hour2, week2, hour_table, week_table)


_KB = 2560


def _fcz_tc(h2f_ref, fcw_ref, whT_ref, wwT_ref, b_ref, eh_ref, ew_ref, o_ref):
    k = pl.program_id(0)

    @pl.when(k == 0)
    def _():
        o_ref[...] = (
            jnp.dot(eh_ref[...], whT_ref[...], preferred_element_type=jnp.float32, precision=lax.Precision.HIGHEST)
            + jnp.dot(ew_ref[...], wwT_ref[...], preferred_element_type=jnp.float32, precision=lax.Precision.HIGHEST)
            + b_ref[...])

    o_ref[...] += lax.dot_general(
        h2f_ref[...], fcw_ref[...], (((1,), (1,)), ((), ())),
        preferred_element_type=jnp.float32, precision=lax.Precision.HIGHEST)


def _fcz(h2f, fc_W, whT, wwT, fc_b2d, emb_h, emb_w):
    return pl.pallas_call(
        _fcz_tc,
        grid=(NN * H2 // _KB,),
        in_specs=[
            pl.BlockSpec((B, _KB), lambda k: (0, k)),
            pl.BlockSpec((ENC, _KB), lambda k: (0, k)),
            pl.BlockSpec((TEMB, ENC), lambda k: (0, 0)),
            pl.BlockSpec((TEMB, ENC), lambda k: (0, 0)),
            pl.BlockSpec((1, ENC), lambda k: (0, 0)),
            pl.BlockSpec((B, TEMB), lambda k: (0, 0)),
            pl.BlockSpec((B, TEMB), lambda k: (0, 0)),
        ],
        out_specs=pl.BlockSpec((B, ENC), lambda k: (0, 0)),
        out_shape=jax.ShapeDtypeStruct((B, ENC), jnp.float32),
    )(h2f, fc_W, whT, wwT, fc_b2d, emb_h, emb_w)


_MB2 = 2560


def _fc2_tc(z_ref, w_ref, b_ref, o_ref):
    o_ref[...] = jnp.maximum(
        lax.dot_general(z_ref[...], w_ref[...], (((1,), (1,)), ((), ())),
                        preferred_element_type=jnp.float32, precision=lax.Precision.HIGHEST)
        + b_ref[...], 0.0)


def _fc2(z656, fc2_W, fc2_b2d):
    kdim = ENC + 2 * TEMB
    return pl.pallas_call(
        _fc2_tc,
        grid=(NN * DEC // _MB2,),
        in_specs=[
            pl.BlockSpec((B, kdim), lambda m: (0, 0)),
            pl.BlockSpec((_MB2, kdim), lambda m: (m, 0)),
            pl.BlockSpec((1, _MB2), lambda m: (0, m)),
        ],
        out_specs=pl.BlockSpec((B, _MB2), lambda m: (0, m)),
        out_shape=jax.ShapeDtypeStruct((B, NN * DEC), jnp.float32),
    )(z656, fc2_W, fc2_b2d)


def _decpre_tc(zd_ref, waT_ref, wbT_ref, b_ref, a_ref, bo_ref):
    zd = zd_ref[...]
    a_ref[...] = jnp.dot(zd, waT_ref[...], preferred_element_type=jnp.float32, precision=lax.Precision.HIGHEST)
    bo_ref[...] = (jnp.dot(zd, wbT_ref[...], preferred_element_type=jnp.float32, precision=lax.Precision.HIGHEST)
                   + b_ref[...])


def _decpre(zdn, waT, wbT, b1_2d):
    return pl.pallas_call(
        _decpre_tc,
        grid=(N // _RB,),
        in_specs=[
            pl.BlockSpec((_RB, DEC), lambda i: (i, 0)),
            pl.BlockSpec((DEC, H1), lambda i: (0, 0)),
            pl.BlockSpec((DEC, H1), lambda i: (0, 0)),
            pl.BlockSpec((1, H1), lambda i: (0, 0)),
        ],
        out_specs=[
            pl.BlockSpec((_RB, H1), lambda i: (i, 0)),
            pl.BlockSpec((_RB, H1), lambda i: (i, 0)),
        ],
        out_shape=[
            jax.ShapeDtypeStruct((N, H1), jnp.float32),
            jax.ShapeDtypeStruct((N, H1), jnp.float32),
        ],
    )(zdn, waT, wbT, b1_2d)


def kernel(x, edge_index, edge_attr, hour, week,
           W1_rel, b1_rel, W1_root, W2_rel, b2_rel, W2_root,
           hour_table, week_table, fc_W, fc_b, fc2_W, fc2_b,
           dec_W1, dec_b1, dec_W2, dec_b2):
    src = edge_index[0]
    dst = edge_index[1]
    zeros = jnp.zeros((ACC_ROWS, F), jnp.float32)

    agg1 = _segsum(x, src, dst, edge_attr, zeros)
    h1 = _lin1(agg1, x, W1_rel.T, W1_root.T, b1_rel[None, :])

    y2, r2 = _lin2pre(h1, W2_rel.T, W2_root.T, b2_rel[None, :])
    agg2 = _segsum(y2, src, dst, edge_attr, zeros)
    h2 = _h2ew(agg2, r2)

    emb_h, emb_w = _emb(hour[:, None], week[:, None], hour_table, week_table)
    z = _fcz(h2.reshape(B, NN * H2), fc_W, fc_W[:, NN * H2:NN * H2 + TEMB].T,
             fc_W[:, NN * H2 + TEMB:].T, fc_b[None, :], emb_h, emb_w)
    z656 = jnp.concatenate([z, emb_h, emb_w], axis=1)
    zd = _fc2(z656, fc2_W, fc2_b[None, :])

    zdn = zd.reshape(N, DEC)
    A, Bmb = _decpre(zdn, dec_W1[:, :DEC].T, dec_W1[:, DEC:].T, dec_b1[None, :])
    w2ext = jnp.concatenate(
        [dec_W2[0], jnp.full((LANES,), dec_b2[0], jnp.float32)])
    pred = _edgedec(A, Bmb, src, dst, w2ext)
    return pred[:, None]


# speedup vs baseline: 1.1469x; 1.1469x over previous
import functools

import jax
import jax.numpy as jnp
from jax import lax
from jax.experimental import pallas as pl
from jax.experimental.pallas import tpu as pltpu
from jax.experimental.pallas import tpu_sc as plsc

B = 32
NN = 1000
N = B * NN
E = 512000
D_IN = 128
H1 = 256
H2 = 128
ENC = 256
DEC = 128
TEMB = 200

NC = 2
NS = 16
LANES = 16
ROUNDS = 2
QTR = N // (NC * ROUNDS)
ACC_ROWS = 8192

F = 128
SEG_CHUNK = 128
DEC_CHUNK = 80


def _sc_mesh():
    return plsc.VectorSubcoreMesh(core_axis_name="c", subcore_axis_name="s",
                                  num_cores=NC, num_subcores=NS)


_DNUMS = lax.GatherDimensionNumbers(
    offset_dims=(), collapsed_slice_dims=(0,), start_index_map=(0,))


def _shuf(v, p):
    return lax.gather(v, p[:, None], _DNUMS, slice_sizes=(1,),
                      mode=lax.GatherScatterMode.PROMISE_IN_BOUNDS)


SEG_STAGE = 2000
FIRE = 128
PEND = 160


def _segsum_body(vals_hbm, src_hbm, dst_hbm, ea_hbm, zeros_hbm, out_hbm,
                 acc_sh, st_src, st_dst, st_ea, p_src, p_dst, p_ea,
                 fire_src, fire_dst, rows_v, sem):
    c = lax.axis_index("c")
    s = lax.axis_index("s")
    ept = E // NS
    rpt = ACC_ROWS // NS
    zero16i = jnp.zeros((LANES,), jnp.int32)
    zero16f = jnp.zeros((LANES,), jnp.float32)
    lanes = lax.iota(jnp.int32, LANES)
    for g in range(PEND // LANES):
        p_src[pl.ds(g * LANES, LANES)] = zero16i
        p_dst[pl.ds(g * LANES, LANES)] = zero16i
        p_ea[pl.ds(g * LANES, LANES)] = zero16f

    def fire():
        for g in range(FIRE // LANES):
            sl = pl.ds(g * LANES, LANES)
            fire_src[sl] = p_src[sl]
            fire_dst[sl] = p_dst[sl]
        pltpu.async_copy(vals_hbm.at[fire_src], rows_v, sem).wait()

        def scale(k, _):
            ea16 = p_ea[pl.ds(k * LANES, LANES)]
            for e in range(LANES):
                m = jnp.full((LANES,), ea16[e], jnp.float32)
                row = k * LANES + e
                for j in range(F // LANES):
                    sl = pl.ds(j * LANES, LANES)
                    rows_v[row, sl] = rows_v[row, sl] * m
            return 0

        lax.fori_loop(0, FIRE // LANES, scale, 0)
        pltpu.sync_copy(rows_v, acc_sh.at[fire_dst], add=True)

    for r in range(ROUNDS):
        base_node = (r * NC + c) * QTR
        pltpu.sync_copy(zeros_hbm.at[pl.ds(s * rpt, rpt)],
                        acc_sh.at[pl.ds(s * rpt, rpt)])
        plsc.subcore_barrier()

        def stage(t, cur):
            ebase = s * ept + t * SEG_STAGE
            pltpu.sync_copy(src_hbm.at[pl.ds(ebase, SEG_STAGE)], st_src)
            pltpu.sync_copy(dst_hbm.at[pl.ds(ebase, SEG_STAGE)], st_dst)
            pltpu.sync_copy(ea_hbm.at[pl.ds(ebase, SEG_STAGE)], st_ea)

            def grp(g, cur):
                sl = pl.ds(g * LANES, LANES)
                local = st_dst[sl] - base_node
                m = (local >= 0) & (local < QTR)
                m_f = jnp.where(m, 1.0, 0.0)
                incl = m_f
                for sh in (1, 2, 4, 8):
                    t = _shuf(incl, jnp.maximum(lanes - sh, 0))
                    incl = incl + jnp.where(lanes >= sh, t, 0.0)
                excl = (incl - m_f).astype(jnp.int32)
                pos = jnp.where(m, cur + excl, PEND + lanes)
                plsc.store_scatter(p_dst, [pos], local)
                plsc.store_scatter(p_src, [pos], st_src[sl])
                plsc.store_scatter(p_ea, [pos], st_ea[sl])
                cur = cur + incl[LANES - 1].astype(jnp.int32)

                @pl.when(cur >= FIRE)
                def _():
                    fire()
                    for a in (p_src, p_dst, p_ea):
                        a[pl.ds(0, LANES)] = a[pl.ds(FIRE, LANES)]

                return jnp.where(cur >= FIRE, cur - FIRE, cur)

            return lax.fori_loop(0, SEG_STAGE // LANES, grp, cur)

        cur = lax.fori_loop(0, ept // SEG_STAGE, stage, jnp.int32(0))
        for g in range(FIRE // LANES):
            sl = pl.ds(g * LANES, LANES)
            keep = (lanes + g * LANES) < cur
            p_ea[sl] = jnp.where(keep, p_ea[sl], 0.0)
        fire()
        plsc.subcore_barrier()
        @pl.when(s < 10)
        def _():
            pltpu.sync_copy(acc_sh.at[pl.ds(s * 800, 800)],
                            out_hbm.at[pl.ds(base_node + s * 800, 800)])
        plsc.subcore_barrier()


def _segsum(vals, src, dst, ea, zeros):
    k = pl.kernel(
        _segsum_body,
        out_type=jax.ShapeDtypeStruct((N, F), jnp.float32),
        mesh=_sc_mesh(),
        compiler_params=pltpu.CompilerParams(needs_layout_passes=False),
        scratch_types=[
            pltpu.VMEM_SHARED((ACC_ROWS, F), jnp.float32),
            pltpu.VMEM((SEG_STAGE,), jnp.int32),
            pltpu.VMEM((SEG_STAGE,), jnp.int32),
            pltpu.VMEM((SEG_STAGE,), jnp.float32),
            pltpu.VMEM((PEND + LANES,), jnp.int32),
            pltpu.VMEM((PEND + LANES,), jnp.int32),
            pltpu.VMEM((PEND + LANES,), jnp.float32),
            pltpu.VMEM((FIRE,), jnp.int32),
            pltpu.VMEM((FIRE,), jnp.int32),
            pltpu.VMEM((FIRE, F), jnp.float32),
            pltpu.SemaphoreType.DMA,
        ],
    )
    return k(vals, src, dst, ea, zeros)


def _edgedec_body(a_hbm, b_hbm, src_hbm, dst_hbm, w2_hbm, out_hbm,
                  idx_s0, idx_d0, idx_s1, idx_d1, ar0, br0, ar1, br1,
                  w2_v, outc, acc2d, sem0, sem1):
    c = lax.axis_index("c")
    s = lax.axis_index("s")
    wid = c * NS + s
    ept = E // (NC * NS)
    n_chunks = ept // DEC_CHUNK
    pltpu.sync_copy(w2_hbm, w2_v)
    b2v = w2_v[pl.ds(H1, LANES)]
    w2sl = [w2_v[pl.ds(j * LANES, LANES)] for j in range(H1 // LANES)]
    lanes = lax.iota(jnp.int32, LANES)

    def issue(i, idx_s, idx_d, ar, br, sem):
        ebase = wid * ept + i * DEC_CHUNK
        pltpu.sync_copy(src_hbm.at[pl.ds(ebase, DEC_CHUNK)], idx_s)
        pltpu.sync_copy(dst_hbm.at[pl.ds(ebase, DEC_CHUNK)], idx_d)
        da = pltpu.async_copy(a_hbm.at[idx_s], ar, sem)
        db = pltpu.async_copy(b_hbm.at[idx_d], br, sem)
        return da, db

    def compute(i, arows, brows):
        def grp(g, _):
            for e in range(LANES):
                row = g * LANES + e
                acc0 = jnp.zeros((LANES,), jnp.float32)
                acc1 = jnp.zeros((LANES,), jnp.float32)
                for j in range(0, H1 // LANES, 2):
                    sl0 = pl.ds(j * LANES, LANES)
                    sl1 = pl.ds((j + 1) * LANES, LANES)
                    t0 = jnp.maximum(arows[row, sl0] + brows[row, sl0], 0.0)
                    t1 = jnp.maximum(arows[row, sl1] + brows[row, sl1], 0.0)
                    acc0 = acc0 + t0 * w2sl[j]
                    acc1 = acc1 + t1 * w2sl[j + 1]
                acc2d[e, :] = acc0 + acc1
            cols = [plsc.load_gather(
                acc2d, [lanes, jnp.full((LANES,), j, jnp.int32)])
                for j in range(LANES)]
            while len(cols) > 1:
                cols = [cols[k] + cols[k + 1] for k in range(0, len(cols), 2)]
            t = cols[0] + b2v
            outc[pl.ds(g * LANES, LANES)] = 1.0 / (1.0 + jnp.exp(-t))
            return 0

        lax.fori_loop(0, DEC_CHUNK // LANES, grp, 0)
        ebase = wid * ept + i * DEC_CHUNK
        pltpu.sync_copy(outc, out_hbm.at[pl.ds(ebase, DEC_CHUNK)])

    d0 = issue(0, idx_s0, idx_d0, ar0, br0, sem0)

    def pipe(h, _):
        i0 = 2 * h
        d1 = issue(i0 + 1, idx_s1, idx_d1, ar1, br1, sem1)
        for d in d0:
            d.wait()
        compute(i0, ar0, br0)

        @pl.when(h < n_chunks // 2 - 1)
        def _():
            issue(i0 + 2, idx_s0, idx_d0, ar0, br0, sem0)

        for d in d1:
            d.wait()
        compute(i0 + 1, ar1, br1)
        return 0

    lax.fori_loop(0, n_chunks // 2, pipe, 0)


def _edgedec(A, Bmb, src, dst, w2ext):
    k = pl.kernel(
        _edgedec_body,
        out_type=jax.ShapeDtypeStruct((E,), jnp.float32),
        mesh=_sc_mesh(),
        compiler_params=pltpu.CompilerParams(needs_layout_passes=False),
        scratch_types=[
            pltpu.VMEM((DEC_CHUNK,), jnp.int32),
            pltpu.VMEM((DEC_CHUNK,), jnp.int32),
            pltpu.VMEM((DEC_CHUNK,), jnp.int32),
            pltpu.VMEM((DEC_CHUNK,), jnp.int32),
            pltpu.VMEM((DEC_CHUNK, H1), jnp.float32),
            pltpu.VMEM((DEC_CHUNK, H1), jnp.float32),
            pltpu.VMEM((DEC_CHUNK, H1), jnp.float32),
            pltpu.VMEM((DEC_CHUNK, H1), jnp.float32),
            pltpu.VMEM((H1 + LANES,), jnp.float32),
            pltpu.VMEM((DEC_CHUNK,), jnp.float32),
            pltpu.VMEM((LANES, LANES), jnp.float32),
            pltpu.SemaphoreType.DMA,
            pltpu.SemaphoreType.DMA,
        ],
    )
    return k(A, Bmb, src, dst, w2ext)


_RB = 800


def _lin1_tc(agg_ref, x_ref, wr_ref, wq_ref, b_ref, o_ref):
    o_ref[...] = jnp.maximum(
        jnp.dot(agg_ref[...], wr_ref[...], preferred_element_type=jnp.float32)
        + jnp.dot(x_ref[...], wq_ref[...], preferred_element_type=jnp.float32)
        + b_ref[...], 0.0)


def _lin1(agg, x, wrT, wqT, b):
    return pl.pallas_call(
        _lin1_tc,
        grid=(N // _RB,),
        in_specs=[
            pl.BlockSpec((_RB, D_IN), lambda i: (i, 0)),
            pl.BlockSpec((_RB, D_IN), lambda i: (i, 0)),
            pl.BlockSpec((D_IN, H1), lambda i: (0, 0)),
            pl.BlockSpec((D_IN, H1), lambda i: (0, 0)),
            pl.BlockSpec((1, H1), lambda i: (0, 0)),
        ],
        out_specs=pl.BlockSpec((_RB, H1), lambda i: (i, 0)),
        out_shape=jax.ShapeDtypeStruct((N, H1), jnp.float32),
    )(agg, x, wrT, wqT, b)


def _lin2pre_tc(h_ref, wr_ref, wq_ref, b_ref, y_ref, r_ref):
    h = h_ref[...]
    y_ref[...] = jnp.dot(h, wr_ref[...], preferred_element_type=jnp.float32)
    r_ref[...] = (jnp.dot(h, wq_ref[...], preferred_element_type=jnp.float32)
                  + b_ref[...])


def _lin2pre(h1, wrT, wqT, b):
    return pl.pallas_call(
        _lin2pre_tc,
        grid=(N // _RB,),
        in_specs=[
            pl.BlockSpec((_RB, H1), lambda i: (i, 0)),
            pl.BlockSpec((H1, H2), lambda i: (0, 0)),
            pl.BlockSpec((H1, H2), lambda i: (0, 0)),
            pl.BlockSpec((1, H2), lambda i: (0, 0)),
        ],
        out_specs=[
            pl.BlockSpec((_RB, H2), lambda i: (i, 0)),
            pl.BlockSpec((_RB, H2), lambda i: (i, 0)),
        ],
        out_shape=[
            jax.ShapeDtypeStruct((N, H2), jnp.float32),
            jax.ShapeDtypeStruct((N, H2), jnp.float32),
        ],
    )(h1, wrT, wqT, b)


def _h2ew_tc(agg_ref, r_ref, o_ref):
    o_ref[...] = jnp.maximum(agg_ref[...] + r_ref[...], 0.0)


def _h2ew(agg2, r2):
    return pl.pallas_call(
        _h2ew_tc,
        grid=(N // _RB,),
        in_specs=[
            pl.BlockSpec((_RB, H2), lambda i: (i, 0)),
            pl.BlockSpec((_RB, H2), lambda i: (i, 0)),
        ],
        out_specs=pl.BlockSpec((_RB, H2), lambda i: (i, 0)),
        out_shape=jax.ShapeDtypeStruct((N, H2), jnp.float32),
    )(agg2, r2)


def _emb_tc(hour_ref, week_ref, ht_ref, wt_ref, eh_ref, ew_ref):
    oh = (hour_ref[...] == lax.broadcasted_iota(jnp.int32, (B, 24), 1)
          ).astype(jnp.float32)
    ow = (week_ref[...] == lax.broadcasted_iota(jnp.int32, (B, 7), 1)
          ).astype(jnp.float32)
    eh_ref[...] = jnp.dot(oh, ht_ref[...], preferred_element_type=jnp.float32)
    ew_ref[...] = jnp.dot(ow, wt_ref[...], preferred_element_type=jnp.float32)


def _emb(hour2, week2, hour_table, week_table):
    return pl.pallas_call(
        _emb_tc,
        out_shape=[
            jax.ShapeDtypeStruct((B, TEMB), jnp.float32),
            jax.ShapeDtypeStruct((B, TEMB), jnp.float32),
        ],
    )(hour2, week2, hour_table, week_table)


_KB = 2560


def _fcz_tc(h2f_ref, fcw_ref, whT_ref, wwT_ref, b_ref, eh_ref, ew_ref, o_ref):
    k = pl.program_id(0)

    @pl.when(k == 0)
    def _():
        o_ref[...] = (
            jnp.dot(eh_ref[...], whT_ref[...], preferred_element_type=jnp.float32)
            + jnp.dot(ew_ref[...], wwT_ref[...], preferred_element_type=jnp.float32)
            + b_ref[...])

    o_ref[...] += lax.dot_general(
        h2f_ref[...], fcw_ref[...], (((1,), (1,)), ((), ())),
        preferred_element_type=jnp.float32)


def _fcz(h2f, fc_W, whT, wwT, fc_b2d, emb_h, emb_w):
    return pl.pallas_call(
        _fcz_tc,
        grid=(NN * H2 // _KB,),
        in_specs=[
            pl.BlockSpec((B, _KB), lambda k: (0, k)),
            pl.BlockSpec((ENC, _KB), lambda k: (0, k)),
            pl.BlockSpec((TEMB, ENC), lambda k: (0, 0)),
            pl.BlockSpec((TEMB, ENC), lambda k: (0, 0)),
            pl.BlockSpec((1, ENC), lambda k: (0, 0)),
            pl.BlockSpec((B, TEMB), lambda k: (0, 0)),
            pl.BlockSpec((B, TEMB), lambda k: (0, 0)),
        ],
        out_specs=pl.BlockSpec((B, ENC), lambda k: (0, 0)),
        out_shape=jax.ShapeDtypeStruct((B, ENC), jnp.float32),
    )(h2f, fc_W, whT, wwT, fc_b2d, emb_h, emb_w)


_MB2 = 2560


def _fc2_tc(z_ref, w_ref, b_ref, o_ref):
    o_ref[...] = jnp.maximum(
        lax.dot_general(z_ref[...], w_ref[...], (((1,), (1,)), ((), ())),
                        preferred_element_type=jnp.float32)
        + b_ref[...], 0.0)


def _fc2(z656, fc2_W, fc2_b2d):
    kdim = ENC + 2 * TEMB
    return pl.pallas_call(
        _fc2_tc,
        grid=(NN * DEC // _MB2,),
        in_specs=[
            pl.BlockSpec((B, kdim), lambda m: (0, 0)),
            pl.BlockSpec((_MB2, kdim), lambda m: (m, 0)),
            pl.BlockSpec((1, _MB2), lambda m: (0, m)),
        ],
        out_specs=pl.BlockSpec((B, _MB2), lambda m: (0, m)),
        out_shape=jax.ShapeDtypeStruct((B, NN * DEC), jnp.float32),
    )(z656, fc2_W, fc2_b2d)


def _decpre_tc(zd_ref, waT_ref, wbT_ref, b_ref, a_ref, bo_ref):
    zd = zd_ref[...]
    a_ref[...] = jnp.dot(zd, waT_ref[...], preferred_element_type=jnp.float32)
    bo_ref[...] = (jnp.dot(zd, wbT_ref[...], preferred_element_type=jnp.float32)
                   + b_ref[...])


def _decpre(zdn, waT, wbT, b1_2d):
    return pl.pallas_call(
        _decpre_tc,
        grid=(N // _RB,),
        in_specs=[
            pl.BlockSpec((_RB, DEC), lambda i: (i, 0)),
            pl.BlockSpec((DEC, H1), lambda i: (0, 0)),
            pl.BlockSpec((DEC, H1), lambda i: (0, 0)),
            pl.BlockSpec((1, H1), lambda i: (0, 0)),
        ],
        out_specs=[
            pl.BlockSpec((_RB, H1), lambda i: (i, 0)),
            pl.BlockSpec((_RB, H1), lambda i: (i, 0)),
        ],
        out_shape=[
            jax.ShapeDtypeStruct((N, H1), jnp.float32),
            jax.ShapeDtypeStruct((N, H1), jnp.float32),
        ],
    )(zdn, waT, wbT, b1_2d)


def kernel(x, edge_index, edge_attr, hour, week,
           W1_rel, b1_rel, W1_root, W2_rel, b2_rel, W2_root,
           hour_table, week_table, fc_W, fc_b, fc2_W, fc2_b,
           dec_W1, dec_b1, dec_W2, dec_b2):
    src = edge_index[0]
    dst = edge_index[1]
    zeros = jnp.zeros((ACC_ROWS, F), jnp.float32)

    agg1 = _segsum(x, src, dst, edge_attr, zeros)
    h1 = _lin1(agg1, x, W1_rel.T, W1_root.T, b1_rel[None, :])

    y2, r2 = _lin2pre(h1, W2_rel.T, W2_root.T, b2_rel[None, :])
    agg2 = _segsum(y2, src, dst, edge_attr, zeros)
    h2 = _h2ew(agg2, r2)

    emb_h, emb_w = _emb(hour[:, None], week[:, None], hour_table, week_table)
    z = _fcz(h2.reshape(B, NN * H2), fc_W, fc_W[:, NN * H2:NN * H2 + TEMB].T,
             fc_W[:, NN * H2 + TEMB:].T, fc_b[None, :], emb_h, emb_w)
    z656 = jnp.concatenate([z, emb_h, emb_w], axis=1)
    zd = _fc2(z656, fc2_W, fc2_b[None, :])

    zdn = zd.reshape(N, DEC)
    A, Bmb = _decpre(zdn, dec_W1[:, :DEC].T, dec_W1[:, DEC:].T, dec_b1[None, :])
    w2ext = jnp.concatenate(
        [dec_W2[0], jnp.full((LANES,), dec_b2[0], jnp.float32)])
    pred = _edgedec(A, Bmb, src, dst, w2ext)
    return pred[:, None]


# speedup vs baseline: 1.1597x; 1.0111x over previous
import functools

import jax
import jax.numpy as jnp
from jax import lax
from jax.experimental import pallas as pl
from jax.experimental.pallas import tpu as pltpu
from jax.experimental.pallas import tpu_sc as plsc

B = 32
NN = 1000
N = B * NN
E = 512000
D_IN = 128
H1 = 256
H2 = 128
ENC = 256
DEC = 128
TEMB = 200

NC = 2
NS = 16
LANES = 16
ROUNDS = 2
QTR = N // (NC * ROUNDS)
ACC_ROWS = 8192

F = 128
SEG_CHUNK = 128
DEC_CHUNK = 80


def _sc_mesh():
    return plsc.VectorSubcoreMesh(core_axis_name="c", subcore_axis_name="s",
                                  num_cores=NC, num_subcores=NS)


_DNUMS = lax.GatherDimensionNumbers(
    offset_dims=(), collapsed_slice_dims=(0,), start_index_map=(0,))


def _shuf(v, p):
    return lax.gather(v, p[:, None], _DNUMS, slice_sizes=(1,),
                      mode=lax.GatherScatterMode.PROMISE_IN_BOUNDS)


SEG_STAGE = 2000
FIRE = 128
PEND = 160


def _segsum_body(vals_hbm, src_hbm, dst_hbm, ea_hbm, zeros_hbm, out_hbm,
                 acc_sh, st_src, st_dst, st_ea, p_src, p_dst, p_ea,
                 fire_src, fire_dst, rows_v, sem):
    c = lax.axis_index("c")
    s = lax.axis_index("s")
    ept = E // NS
    rpt = ACC_ROWS // NS
    zero16i = jnp.zeros((LANES,), jnp.int32)
    zero16f = jnp.zeros((LANES,), jnp.float32)
    lanes = lax.iota(jnp.int32, LANES)
    for g in range(PEND // LANES):
        p_src[pl.ds(g * LANES, LANES)] = zero16i
        p_dst[pl.ds(g * LANES, LANES)] = zero16i
        p_ea[pl.ds(g * LANES, LANES)] = zero16f

    def fire():
        for g in range(FIRE // LANES):
            sl = pl.ds(g * LANES, LANES)
            fire_src[sl] = p_src[sl]
            fire_dst[sl] = p_dst[sl]
        pltpu.async_copy(vals_hbm.at[fire_src], rows_v, sem).wait()

        def scale(k, _):
            ea16 = p_ea[pl.ds(k * LANES, LANES)]
            for e in range(LANES):
                m = jnp.full((LANES,), ea16[e], jnp.float32)
                row = k * LANES + e
                for j in range(F // LANES):
                    sl = pl.ds(j * LANES, LANES)
                    rows_v[row, sl] = rows_v[row, sl] * m
            return 0

        lax.fori_loop(0, FIRE // LANES, scale, 0)
        pltpu.sync_copy(rows_v, acc_sh.at[fire_dst], add=True)

    for r in range(ROUNDS):
        base_node = (r * NC + c) * QTR
        pltpu.sync_copy(zeros_hbm.at[pl.ds(s * rpt, rpt)],
                        acc_sh.at[pl.ds(s * rpt, rpt)])
        plsc.subcore_barrier()

        def stage(t, cur):
            ebase = s * ept + t * SEG_STAGE
            pltpu.sync_copy(src_hbm.at[pl.ds(ebase, SEG_STAGE)], st_src)
            pltpu.sync_copy(dst_hbm.at[pl.ds(ebase, SEG_STAGE)], st_dst)
            pltpu.sync_copy(ea_hbm.at[pl.ds(ebase, SEG_STAGE)], st_ea)

            def grp(g, cur):
                sl = pl.ds(g * LANES, LANES)
                local = st_dst[sl] - base_node
                m = (local >= 0) & (local < QTR)
                m_f = jnp.where(m, 1.0, 0.0)
                incl = m_f
                for sh in (1, 2, 4, 8):
                    t = _shuf(incl, jnp.maximum(lanes - sh, 0))
                    incl = incl + jnp.where(lanes >= sh, t, 0.0)
                excl = (incl - m_f).astype(jnp.int32)
                pos = jnp.where(m, cur + excl, PEND + lanes)
                plsc.store_scatter(p_dst, [pos], local)
                plsc.store_scatter(p_src, [pos], st_src[sl])
                plsc.store_scatter(p_ea, [pos], st_ea[sl])
                cur = cur + incl[LANES - 1].astype(jnp.int32)

                @pl.when(cur >= FIRE)
                def _():
                    fire()
                    for a in (p_src, p_dst, p_ea):
                        a[pl.ds(0, LANES)] = a[pl.ds(FIRE, LANES)]

                return jnp.where(cur >= FIRE, cur - FIRE, cur)

            return lax.fori_loop(0, SEG_STAGE // LANES, grp, cur)

        cur = lax.fori_loop(0, ept // SEG_STAGE, stage, jnp.int32(0))
        for g in range(FIRE // LANES):
            sl = pl.ds(g * LANES, LANES)
            keep = (lanes + g * LANES) < cur
            p_ea[sl] = jnp.where(keep, p_ea[sl], 0.0)
        fire()
        plsc.subcore_barrier()
        @pl.when(s < 10)
        def _():
            pltpu.sync_copy(acc_sh.at[pl.ds(s * 800, 800)],
                            out_hbm.at[pl.ds(base_node + s * 800, 800)])
        plsc.subcore_barrier()


def _segsum(vals, src, dst, ea, zeros):
    k = pl.kernel(
        _segsum_body,
        out_type=jax.ShapeDtypeStruct((N, F), jnp.float32),
        mesh=_sc_mesh(),
        compiler_params=pltpu.CompilerParams(needs_layout_passes=False),
        scratch_types=[
            pltpu.VMEM_SHARED((ACC_ROWS, F), jnp.float32),
            pltpu.VMEM((SEG_STAGE,), jnp.int32),
            pltpu.VMEM((SEG_STAGE,), jnp.int32),
            pltpu.VMEM((SEG_STAGE,), jnp.float32),
            pltpu.VMEM((PEND + LANES,), jnp.int32),
            pltpu.VMEM((PEND + LANES,), jnp.int32),
            pltpu.VMEM((PEND + LANES,), jnp.float32),
            pltpu.VMEM((FIRE,), jnp.int32),
            pltpu.VMEM((FIRE,), jnp.int32),
            pltpu.VMEM((FIRE, F), jnp.float32),
            pltpu.SemaphoreType.DMA,
        ],
    )
    return k(vals, src, dst, ea, zeros)


def _edgedec_body(a_hbm, b_hbm, src_hbm, dst_hbm, w2_hbm, out_hbm,
                  idx_s0, idx_d0, idx_s1, idx_d1, ar0, br0, ar1, br1,
                  w2_v, outc, acc2d, sem0, sem1):
    c = lax.axis_index("c")
    s = lax.axis_index("s")
    wid = c * NS + s
    ept = E // (NC * NS)
    n_chunks = ept // DEC_CHUNK
    pltpu.sync_copy(w2_hbm, w2_v)
    b2v = w2_v[pl.ds(H1, LANES)]
    w2sl = [w2_v[pl.ds(j * LANES, LANES)] for j in range(H1 // LANES)]
    lanes = lax.iota(jnp.int32, LANES)

    def issue(i, idx_s, idx_d, ar, br, sem):
        ebase = wid * ept + i * DEC_CHUNK
        pltpu.sync_copy(src_hbm.at[pl.ds(ebase, DEC_CHUNK)], idx_s)
        pltpu.sync_copy(dst_hbm.at[pl.ds(ebase, DEC_CHUNK)], idx_d)
        da = pltpu.async_copy(a_hbm.at[idx_s], ar, sem)
        db = pltpu.async_copy(b_hbm.at[idx_d], br, sem)
        return da, db

    def compute(i, arows, brows):
        def grp(g, _):
            for e in range(LANES):
                row = g * LANES + e
                acc0 = jnp.zeros((LANES,), jnp.float32)
                acc1 = jnp.zeros((LANES,), jnp.float32)
                for j in range(0, H1 // LANES, 2):
                    sl0 = pl.ds(j * LANES, LANES)
                    sl1 = pl.ds((j + 1) * LANES, LANES)
                    t0 = jnp.maximum(arows[row, sl0] + brows[row, sl0], 0.0)
                    t1 = jnp.maximum(arows[row, sl1] + brows[row, sl1], 0.0)
                    acc0 = acc0 + t0 * w2sl[j]
                    acc1 = acc1 + t1 * w2sl[j + 1]
                acc2d[e, :] = acc0 + acc1
            cols = [plsc.load_gather(
                acc2d, [lanes, jnp.full((LANES,), j, jnp.int32)])
                for j in range(LANES)]
            while len(cols) > 1:
                cols = [cols[k] + cols[k + 1] for k in range(0, len(cols), 2)]
            t = cols[0] + b2v
            outc[pl.ds(g * LANES, LANES)] = 1.0 / (1.0 + jnp.exp(-t))
            return 0

        lax.fori_loop(0, DEC_CHUNK // LANES, grp, 0)
        ebase = wid * ept + i * DEC_CHUNK
        pltpu.sync_copy(outc, out_hbm.at[pl.ds(ebase, DEC_CHUNK)])

    d0 = issue(0, idx_s0, idx_d0, ar0, br0, sem0)

    def pipe(h, _):
        i0 = 2 * h
        d1 = issue(i0 + 1, idx_s1, idx_d1, ar1, br1, sem1)
        for d in d0:
            d.wait()
        compute(i0, ar0, br0)

        @pl.when(h < n_chunks // 2 - 1)
        def _():
            issue(i0 + 2, idx_s0, idx_d0, ar0, br0, sem0)

        for d in d1:
            d.wait()
        compute(i0 + 1, ar1, br1)
        return 0

    lax.fori_loop(0, n_chunks // 2, pipe, 0)


def _edgedec(A, Bmb, src, dst, w2ext):
    k = pl.kernel(
        _edgedec_body,
        out_type=jax.ShapeDtypeStruct((E,), jnp.float32),
        mesh=_sc_mesh(),
        compiler_params=pltpu.CompilerParams(needs_layout_passes=False),
        scratch_types=[
            pltpu.VMEM((DEC_CHUNK,), jnp.int32),
            pltpu.VMEM((DEC_CHUNK,), jnp.int32),
            pltpu.VMEM((DEC_CHUNK,), jnp.int32),
            pltpu.VMEM((DEC_CHUNK,), jnp.int32),
            pltpu.VMEM((DEC_CHUNK, H1), jnp.float32),
            pltpu.VMEM((DEC_CHUNK, H1), jnp.float32),
            pltpu.VMEM((DEC_CHUNK, H1), jnp.float32),
            pltpu.VMEM((DEC_CHUNK, H1), jnp.float32),
            pltpu.VMEM((H1 + LANES,), jnp.float32),
            pltpu.VMEM((DEC_CHUNK,), jnp.float32),
            pltpu.VMEM((LANES, LANES), jnp.float32),
            pltpu.SemaphoreType.DMA,
            pltpu.SemaphoreType.DMA,
        ],
    )
    return k(A, Bmb, src, dst, w2ext)


_RB = 800


def _lin1_tc(agg_ref, x_ref, wr_ref, wq_ref, b_ref, o_ref):
    o_ref[...] = jnp.maximum(
        jnp.dot(agg_ref[...], wr_ref[...], preferred_element_type=jnp.float32)
        + jnp.dot(x_ref[...], wq_ref[...], preferred_element_type=jnp.float32)
        + b_ref[...], 0.0)


def _lin1(agg, x, wrT, wqT, b):
    return pl.pallas_call(
        _lin1_tc,
        grid=(N // _RB,),
        in_specs=[
            pl.BlockSpec((_RB, D_IN), lambda i: (i, 0)),
            pl.BlockSpec((_RB, D_IN), lambda i: (i, 0)),
            pl.BlockSpec((D_IN, H1), lambda i: (0, 0)),
            pl.BlockSpec((D_IN, H1), lambda i: (0, 0)),
            pl.BlockSpec((1, H1), lambda i: (0, 0)),
        ],
        out_specs=pl.BlockSpec((_RB, H1), lambda i: (i, 0)),
        out_shape=jax.ShapeDtypeStruct((N, H1), jnp.float32),
    )(agg, x, wrT, wqT, b)


def _lin2pre_tc(h_ref, wr_ref, wq_ref, b_ref, y_ref, r_ref):
    h = h_ref[...]
    y_ref[...] = jnp.dot(h, wr_ref[...], preferred_element_type=jnp.float32)
    r_ref[...] = (jnp.dot(h, wq_ref[...], preferred_element_type=jnp.float32)
                  + b_ref[...])


def _lin2pre(h1, wrT, wqT, b):
    return pl.pallas_call(
        _lin2pre_tc,
        grid=(N // _RB,),
        in_specs=[
            pl.BlockSpec((_RB, H1), lambda i: (i, 0)),
            pl.BlockSpec((H1, H2), lambda i: (0, 0)),
            pl.BlockSpec((H1, H2), lambda i: (0, 0)),
            pl.BlockSpec((1, H2), lambda i: (0, 0)),
        ],
        out_specs=[
            pl.BlockSpec((_RB, H2), lambda i: (i, 0)),
            pl.BlockSpec((_RB, H2), lambda i: (i, 0)),
        ],
        out_shape=[
            jax.ShapeDtypeStruct((N, H2), jnp.float32),
            jax.ShapeDtypeStruct((N, H2), jnp.float32),
        ],
    )(h1, wrT, wqT, b)


def _emb_tc(hour_ref, week_ref, ht_ref, wt_ref, eh_ref, ew_ref):
    oh = (hour_ref[...] == lax.broadcasted_iota(jnp.int32, (B, 24), 1)
          ).astype(jnp.float32)
    ow = (week_ref[...] == lax.broadcasted_iota(jnp.int32, (B, 7), 1)
          ).astype(jnp.float32)
    eh_ref[...] = jnp.dot(oh, ht_ref[...], preferred_element_type=jnp.float32)
    ew_ref[...] = jnp.dot(ow, wt_ref[...], preferred_element_type=jnp.float32)


def _emb(hour2, week2, hour_table, week_table):
    return pl.pallas_call(
        _emb_tc,
        out_shape=[
            jax.ShapeDtypeStruct((B, TEMB), jnp.float32),
            jax.ShapeDtypeStruct((B, TEMB), jnp.float32),
        ],
    )(hour2, week2, hour_table, week_table)


_KB = 2560


def _fcz_tc(agg_ref, r_ref, fcw_ref, whT_ref, wwT_ref, b_ref, eh_ref, ew_ref,
            o_ref):
    k = pl.program_id(0)

    @pl.when(k == 0)
    def _():
        o_ref[...] = (
            jnp.dot(eh_ref[...], whT_ref[...], preferred_element_type=jnp.float32)
            + jnp.dot(ew_ref[...], wwT_ref[...], preferred_element_type=jnp.float32)
            + b_ref[...])

    h2f = jnp.maximum(agg_ref[...] + r_ref[...], 0.0)
    o_ref[...] += lax.dot_general(
        h2f, fcw_ref[...], (((1,), (1,)), ((), ())),
        preferred_element_type=jnp.float32)


def _fcz(agg2f, r2f, fc_W, whT, wwT, fc_b2d, emb_h, emb_w):
    return pl.pallas_call(
        _fcz_tc,
        grid=(NN * H2 // _KB,),
        in_specs=[
            pl.BlockSpec((B, _KB), lambda k: (0, k)),
            pl.BlockSpec((B, _KB), lambda k: (0, k)),
            pl.BlockSpec((ENC, _KB), lambda k: (0, k)),
            pl.BlockSpec((TEMB, ENC), lambda k: (0, 0)),
            pl.BlockSpec((TEMB, ENC), lambda k: (0, 0)),
            pl.BlockSpec((1, ENC), lambda k: (0, 0)),
            pl.BlockSpec((B, TEMB), lambda k: (0, 0)),
            pl.BlockSpec((B, TEMB), lambda k: (0, 0)),
        ],
        out_specs=pl.BlockSpec((B, ENC), lambda k: (0, 0)),
        out_shape=jax.ShapeDtypeStruct((B, ENC), jnp.float32),
    )(agg2f, r2f, fc_W, whT, wwT, fc_b2d, emb_h, emb_w)


_MB2 = 2560


def _fc2_tc(z_ref, w_ref, b_ref, o_ref):
    o_ref[...] = jnp.maximum(
        lax.dot_general(z_ref[...], w_ref[...], (((1,), (1,)), ((), ())),
                        preferred_element_type=jnp.float32)
        + b_ref[...], 0.0)


def _fc2(z656, fc2_W, fc2_b2d):
    kdim = ENC + 2 * TEMB
    return pl.pallas_call(
        _fc2_tc,
        grid=(NN * DEC // _MB2,),
        in_specs=[
            pl.BlockSpec((B, kdim), lambda m: (0, 0)),
            pl.BlockSpec((_MB2, kdim), lambda m: (m, 0)),
            pl.BlockSpec((1, _MB2), lambda m: (0, m)),
        ],
        out_specs=pl.BlockSpec((B, _MB2), lambda m: (0, m)),
        out_shape=jax.ShapeDtypeStruct((B, NN * DEC), jnp.float32),
    )(z656, fc2_W, fc2_b2d)


def _decpre_tc(zd_ref, waT_ref, wbT_ref, b_ref, a_ref, bo_ref):
    zd = zd_ref[...]
    a_ref[...] = jnp.dot(zd, waT_ref[...], preferred_element_type=jnp.float32)
    bo_ref[...] = (jnp.dot(zd, wbT_ref[...], preferred_element_type=jnp.float32)
                   + b_ref[...])


def _decpre(zdn, waT, wbT, b1_2d):
    return pl.pallas_call(
        _decpre_tc,
        grid=(N // _RB,),
        in_specs=[
            pl.BlockSpec((_RB, DEC), lambda i: (i, 0)),
            pl.BlockSpec((DEC, H1), lambda i: (0, 0)),
            pl.BlockSpec((DEC, H1), lambda i: (0, 0)),
            pl.BlockSpec((1, H1), lambda i: (0, 0)),
        ],
        out_specs=[
            pl.BlockSpec((_RB, H1), lambda i: (i, 0)),
            pl.BlockSpec((_RB, H1), lambda i: (i, 0)),
        ],
        out_shape=[
            jax.ShapeDtypeStruct((N, H1), jnp.float32),
            jax.ShapeDtypeStruct((N, H1), jnp.float32),
        ],
    )(zdn, waT, wbT, b1_2d)


def kernel(x, edge_index, edge_attr, hour, week,
           W1_rel, b1_rel, W1_root, W2_rel, b2_rel, W2_root,
           hour_table, week_table, fc_W, fc_b, fc2_W, fc2_b,
           dec_W1, dec_b1, dec_W2, dec_b2):
    src = edge_index[0]
    dst = edge_index[1]
    zeros = jnp.zeros((ACC_ROWS, F), jnp.float32)

    agg1 = _segsum(x, src, dst, edge_attr, zeros)
    h1 = _lin1(agg1, x, W1_rel.T, W1_root.T, b1_rel[None, :])

    y2, r2 = _lin2pre(h1, W2_rel.T, W2_root.T, b2_rel[None, :])
    agg2 = _segsum(y2, src, dst, edge_attr, zeros)

    emb_h, emb_w = _emb(hour[:, None], week[:, None], hour_table, week_table)
    z = _fcz(agg2.reshape(B, NN * H2), r2.reshape(B, NN * H2), fc_W,
             fc_W[:, NN * H2:NN * H2 + TEMB].T,
             fc_W[:, NN * H2 + TEMB:].T, fc_b[None, :], emb_h, emb_w)
    z656 = jnp.concatenate([z, emb_h, emb_w], axis=1)
    zd = _fc2(z656, fc2_W, fc2_b[None, :])

    zdn = zd.reshape(N, DEC)
    A, Bmb = _decpre(zdn, dec_W1[:, :DEC].T, dec_W1[:, DEC:].T, dec_b1[None, :])
    w2ext = jnp.concatenate(
        [dec_W2[0], jnp.full((LANES,), dec_b2[0], jnp.float32)])
    pred = _edgedec(A, Bmb, src, dst, w2ext)
    return pred[:, None]


# speedup vs baseline: 1.1601x; 1.0003x over previous
import jax
import jax.numpy as jnp
from jax import lax
from jax.experimental import pallas as pl
from jax.experimental.pallas import tpu as pltpu
from jax.experimental.pallas import tpu_sc as plsc

B = 32
NN = 1000
N = B * NN
E = 512000
D_IN = 128
H1 = 256
H2 = 128
ENC = 256
DEC = 128
TEMB = 200

NC = 2
NS = 16
LANES = 16
ROUNDS = 2
QTR = N // (NC * ROUNDS)
ACC_ROWS = 8192

F = 128
DEC_CHUNK = 80


def _sc_mesh():
    return plsc.VectorSubcoreMesh(core_axis_name="c", subcore_axis_name="s",
                                  num_cores=NC, num_subcores=NS)


_DNUMS = lax.GatherDimensionNumbers(
    offset_dims=(), collapsed_slice_dims=(0,), start_index_map=(0,))


def _shuf(v, p):
    return lax.gather(v, p[:, None], _DNUMS, slice_sizes=(1,),
                      mode=lax.GatherScatterMode.PROMISE_IN_BOUNDS)


SEG_STAGE = 2000
FIRE = 128
PEND = 160


def _segsum_body(vals_hbm, src_hbm, dst_hbm, ea_hbm, zeros_hbm, out_hbm,
                 acc_sh, st_src, st_dst, st_ea, p_src, p_dst, p_ea,
                 fire_src, fire_dst, rows_v, sem):
    c = lax.axis_index("c")
    s = lax.axis_index("s")
    ept = E // NS
    rpt = ACC_ROWS // NS
    zero16i = jnp.zeros((LANES,), jnp.int32)
    zero16f = jnp.zeros((LANES,), jnp.float32)
    lanes = lax.iota(jnp.int32, LANES)
    for g in range(PEND // LANES):
        p_src[pl.ds(g * LANES, LANES)] = zero16i
        p_dst[pl.ds(g * LANES, LANES)] = zero16i
        p_ea[pl.ds(g * LANES, LANES)] = zero16f

    def fire():
        for g in range(FIRE // LANES):
            sl = pl.ds(g * LANES, LANES)
            fire_src[sl] = p_src[sl]
            fire_dst[sl] = p_dst[sl]
        pltpu.async_copy(vals_hbm.at[fire_src], rows_v, sem).wait()

        def scale(k, _):
            ea16 = p_ea[pl.ds(k * LANES, LANES)]
            for e in range(LANES):
                m = jnp.full((LANES,), ea16[e], jnp.float32)
                row = k * LANES + e
                for j in range(F // LANES):
                    sl = pl.ds(j * LANES, LANES)
                    rows_v[row, sl] = rows_v[row, sl] * m
            return 0

        lax.fori_loop(0, FIRE // LANES, scale, 0)
        pltpu.sync_copy(rows_v, acc_sh.at[fire_dst], add=True)

    for r in range(ROUNDS):
        base_node = (r * NC + c) * QTR
        pltpu.sync_copy(zeros_hbm.at[pl.ds(s * rpt, rpt)],
                        acc_sh.at[pl.ds(s * rpt, rpt)])
        plsc.subcore_barrier()

        def stage(t, cur):
            ebase = s * ept + t * SEG_STAGE
            pltpu.sync_copy(src_hbm.at[pl.ds(ebase, SEG_STAGE)], st_src)
            pltpu.sync_copy(dst_hbm.at[pl.ds(ebase, SEG_STAGE)], st_dst)
            pltpu.sync_copy(ea_hbm.at[pl.ds(ebase, SEG_STAGE)], st_ea)

            def grp(g, cur):
                sl = pl.ds(g * LANES, LANES)
                local = st_dst[sl] - base_node
                m = (local >= 0) & (local < QTR)
                m_f = jnp.where(m, 1.0, 0.0)
                incl = m_f
                for sh in (1, 2, 4, 8):
                    t = _shuf(incl, jnp.maximum(lanes - sh, 0))
                    incl = incl + jnp.where(lanes >= sh, t, 0.0)
                excl = (incl - m_f).astype(jnp.int32)
                pos = jnp.where(m, cur + excl, PEND + lanes)
                plsc.store_scatter(p_dst, [pos], local)
                plsc.store_scatter(p_src, [pos], st_src[sl])
                plsc.store_scatter(p_ea, [pos], st_ea[sl])
                cur = cur + incl[LANES - 1].astype(jnp.int32)

                @pl.when(cur >= FIRE)
                def _():
                    fire()
                    for a in (p_src, p_dst, p_ea):
                        a[pl.ds(0, LANES)] = a[pl.ds(FIRE, LANES)]

                return jnp.where(cur >= FIRE, cur - FIRE, cur)

            return lax.fori_loop(0, SEG_STAGE // LANES, grp, cur)

        cur = lax.fori_loop(0, ept // SEG_STAGE, stage, jnp.int32(0))
        for g in range(FIRE // LANES):
            sl = pl.ds(g * LANES, LANES)
            keep = (lanes + g * LANES) < cur
            p_ea[sl] = jnp.where(keep, p_ea[sl], 0.0)
        fire()
        plsc.subcore_barrier()
        @pl.when(s < 10)
        def _():
            pltpu.sync_copy(acc_sh.at[pl.ds(s * 800, 800)],
                            out_hbm.at[pl.ds(base_node + s * 800, 800)])
        plsc.subcore_barrier()


def _segsum(vals, src, dst, ea, zeros):
    k = pl.kernel(
        _segsum_body,
        out_type=jax.ShapeDtypeStruct((N, F), jnp.float32),
        mesh=_sc_mesh(),
        compiler_params=pltpu.CompilerParams(needs_layout_passes=False),
        scratch_types=[
            pltpu.VMEM_SHARED((ACC_ROWS, F), jnp.float32),
            pltpu.VMEM((SEG_STAGE,), jnp.int32),
            pltpu.VMEM((SEG_STAGE,), jnp.int32),
            pltpu.VMEM((SEG_STAGE,), jnp.float32),
            pltpu.VMEM((PEND + LANES,), jnp.int32),
            pltpu.VMEM((PEND + LANES,), jnp.int32),
            pltpu.VMEM((PEND + LANES,), jnp.float32),
            pltpu.VMEM((FIRE,), jnp.int32),
            pltpu.VMEM((FIRE,), jnp.int32),
            pltpu.VMEM((FIRE, F), jnp.float32),
            pltpu.SemaphoreType.DMA,
        ],
    )
    return k(vals, src, dst, ea, zeros)


def _edgedec_body(a_hbm, b_hbm, src_hbm, dst_hbm, w2_hbm, out_hbm,
                  idx_s0, idx_d0, idx_s1, idx_d1, ar0, br0, ar1, br1,
                  w2_v, outc, acc2d, sem0, sem1):
    c = lax.axis_index("c")
    s = lax.axis_index("s")
    wid = c * NS + s
    ept = E // (NC * NS)
    n_chunks = ept // DEC_CHUNK
    pltpu.sync_copy(w2_hbm, w2_v)
    b2v = w2_v[pl.ds(H1, LANES)]
    w2sl = [w2_v[pl.ds(j * LANES, LANES)] for j in range(H1 // LANES)]
    lanes = lax.iota(jnp.int32, LANES)

    def issue(i, idx_s, idx_d, ar, br, sem):
        ebase = wid * ept + i * DEC_CHUNK
        pltpu.sync_copy(src_hbm.at[pl.ds(ebase, DEC_CHUNK)], idx_s)
        pltpu.sync_copy(dst_hbm.at[pl.ds(ebase, DEC_CHUNK)], idx_d)
        da = pltpu.async_copy(a_hbm.at[idx_s], ar, sem)
        db = pltpu.async_copy(b_hbm.at[idx_d], br, sem)
        return da, db

    def compute(i, arows, brows):
        def grp(g, _):
            for e in range(LANES):
                row = g * LANES + e
                acc0 = jnp.zeros((LANES,), jnp.float32)
                acc1 = jnp.zeros((LANES,), jnp.float32)
                for j in range(0, H1 // LANES, 2):
                    sl0 = pl.ds(j * LANES, LANES)
                    sl1 = pl.ds((j + 1) * LANES, LANES)
                    t0 = jnp.maximum(arows[row, sl0] + brows[row, sl0], 0.0)
                    t1 = jnp.maximum(arows[row, sl1] + brows[row, sl1], 0.0)
                    acc0 = acc0 + t0 * w2sl[j]
                    acc1 = acc1 + t1 * w2sl[j + 1]
                acc2d[e, :] = acc0 + acc1
            cols = [plsc.load_gather(
                acc2d, [lanes, jnp.full((LANES,), j, jnp.int32)])
                for j in range(LANES)]
            while len(cols) > 1:
                cols = [cols[k] + cols[k + 1] for k in range(0, len(cols), 2)]
            t = cols[0] + b2v
            outc[pl.ds(g * LANES, LANES)] = 1.0 / (1.0 + jnp.exp(-t))
            return 0

        lax.fori_loop(0, DEC_CHUNK // LANES, grp, 0)
        ebase = wid * ept + i * DEC_CHUNK
        pltpu.sync_copy(outc, out_hbm.at[pl.ds(ebase, DEC_CHUNK)])

    d0 = issue(0, idx_s0, idx_d0, ar0, br0, sem0)

    def pipe(h, _):
        i0 = 2 * h
        d1 = issue(i0 + 1, idx_s1, idx_d1, ar1, br1, sem1)
        for d in d0:
            d.wait()
        compute(i0, ar0, br0)

        @pl.when(h < n_chunks // 2 - 1)
        def _():
            issue(i0 + 2, idx_s0, idx_d0, ar0, br0, sem0)

        for d in d1:
            d.wait()
        compute(i0 + 1, ar1, br1)
        return 0

    lax.fori_loop(0, n_chunks // 2, pipe, 0)


def _edgedec(A, Bmb, src, dst, w2ext):
    k = pl.kernel(
        _edgedec_body,
        out_type=jax.ShapeDtypeStruct((E,), jnp.float32),
        mesh=_sc_mesh(),
        compiler_params=pltpu.CompilerParams(needs_layout_passes=False),
        scratch_types=[
            pltpu.VMEM((DEC_CHUNK,), jnp.int32),
            pltpu.VMEM((DEC_CHUNK,), jnp.int32),
            pltpu.VMEM((DEC_CHUNK,), jnp.int32),
            pltpu.VMEM((DEC_CHUNK,), jnp.int32),
            pltpu.VMEM((DEC_CHUNK, H1), jnp.float32),
            pltpu.VMEM((DEC_CHUNK, H1), jnp.float32),
            pltpu.VMEM((DEC_CHUNK, H1), jnp.float32),
            pltpu.VMEM((DEC_CHUNK, H1), jnp.float32),
            pltpu.VMEM((H1 + LANES,), jnp.float32),
            pltpu.VMEM((DEC_CHUNK,), jnp.float32),
            pltpu.VMEM((LANES, LANES), jnp.float32),
            pltpu.SemaphoreType.DMA,
            pltpu.SemaphoreType.DMA,
        ],
    )
    return k(A, Bmb, src, dst, w2ext)


_RB = 800


def _lin1_tc(agg_ref, x_ref, wr_ref, wq_ref, b_ref, o_ref):
    o_ref[...] = jnp.maximum(
        jnp.dot(agg_ref[...], wr_ref[...], preferred_element_type=jnp.float32)
        + jnp.dot(x_ref[...], wq_ref[...], preferred_element_type=jnp.float32)
        + b_ref[...], 0.0)


def _lin1(agg, x, wrT, wqT, b):
    return pl.pallas_call(
        _lin1_tc,
        grid=(N // _RB,),
        in_specs=[
            pl.BlockSpec((_RB, D_IN), lambda i: (i, 0)),
            pl.BlockSpec((_RB, D_IN), lambda i: (i, 0)),
            pl.BlockSpec((D_IN, H1), lambda i: (0, 0)),
            pl.BlockSpec((D_IN, H1), lambda i: (0, 0)),
            pl.BlockSpec((1, H1), lambda i: (0, 0)),
        ],
        out_specs=pl.BlockSpec((_RB, H1), lambda i: (i, 0)),
        out_shape=jax.ShapeDtypeStruct((N, H1), jnp.float32),
    )(agg, x, wrT, wqT, b)


def _lin2pre_tc(h_ref, wr_ref, wq_ref, b_ref, y_ref, r_ref):
    h = h_ref[...]
    y_ref[...] = jnp.dot(h, wr_ref[...], preferred_element_type=jnp.float32)
    r_ref[...] = (jnp.dot(h, wq_ref[...], preferred_element_type=jnp.float32)
                  + b_ref[...])


def _lin2pre(h1, wrT, wqT, b):
    return pl.pallas_call(
        _lin2pre_tc,
        grid=(N // _RB,),
        in_specs=[
            pl.BlockSpec((_RB, H1), lambda i: (i, 0)),
            pl.BlockSpec((H1, H2), lambda i: (0, 0)),
            pl.BlockSpec((H1, H2), lambda i: (0, 0)),
            pl.BlockSpec((1, H2), lambda i: (0, 0)),
        ],
        out_specs=[
            pl.BlockSpec((_RB, H2), lambda i: (i, 0)),
            pl.BlockSpec((_RB, H2), lambda i: (i, 0)),
        ],
        out_shape=[
            jax.ShapeDtypeStruct((N, H2), jnp.float32),
            jax.ShapeDtypeStruct((N, H2), jnp.float32),
        ],
    )(h1, wrT, wqT, b)


def _emb_tc(hour_ref, week_ref, ht_ref, wt_ref, eh_ref, ew_ref):
    oh = (hour_ref[...] == lax.broadcasted_iota(jnp.int32, (B, 24), 1)
          ).astype(jnp.float32)
    ow = (week_ref[...] == lax.broadcasted_iota(jnp.int32, (B, 7), 1)
          ).astype(jnp.float32)
    eh_ref[...] = jnp.dot(oh, ht_ref[...], preferred_element_type=jnp.float32)
    ew_ref[...] = jnp.dot(ow, wt_ref[...], preferred_element_type=jnp.float32)


def _emb(hour2, week2, hour_table, week_table):
    return pl.pallas_call(
        _emb_tc,
        out_shape=[
            jax.ShapeDtypeStruct((B, TEMB), jnp.float32),
            jax.ShapeDtypeStruct((B, TEMB), jnp.float32),
        ],
    )(hour2, week2, hour_table, week_table)


_KB = 2560


def _fcz_tc(agg_ref, r_ref, fcw_ref, whT_ref, wwT_ref, b_ref, eh_ref, ew_ref,
            o_ref):
    k = pl.program_id(0)

    @pl.when(k == 0)
    def _():
        o_ref[...] = (
            jnp.dot(eh_ref[...], whT_ref[...], preferred_element_type=jnp.float32)
            + jnp.dot(ew_ref[...], wwT_ref[...], preferred_element_type=jnp.float32)
            + b_ref[...])

    h2f = jnp.maximum(agg_ref[...] + r_ref[...], 0.0)
    o_ref[...] += lax.dot_general(
        h2f, fcw_ref[...], (((1,), (1,)), ((), ())),
        preferred_element_type=jnp.float32)


def _fcz(agg2f, r2f, fc_W, whT, wwT, fc_b2d, emb_h, emb_w):
    return pl.pallas_call(
        _fcz_tc,
        grid=(NN * H2 // _KB,),
        in_specs=[
            pl.BlockSpec((B, _KB), lambda k: (0, k)),
            pl.BlockSpec((B, _KB), lambda k: (0, k)),
            pl.BlockSpec((ENC, _KB), lambda k: (0, k)),
            pl.BlockSpec((TEMB, ENC), lambda k: (0, 0)),
            pl.BlockSpec((TEMB, ENC), lambda k: (0, 0)),
            pl.BlockSpec((1, ENC), lambda k: (0, 0)),
            pl.BlockSpec((B, TEMB), lambda k: (0, 0)),
            pl.BlockSpec((B, TEMB), lambda k: (0, 0)),
        ],
        out_specs=pl.BlockSpec((B, ENC), lambda k: (0, 0)),
        out_shape=jax.ShapeDtypeStruct((B, ENC), jnp.float32),
    )(agg2f, r2f, fc_W, whT, wwT, fc_b2d, emb_h, emb_w)


_MB2 = 2560


def _fc2_tc(z_ref, w_ref, b_ref, o_ref):
    o_ref[...] = jnp.maximum(
        lax.dot_general(z_ref[...], w_ref[...], (((1,), (1,)), ((), ())),
                        preferred_element_type=jnp.float32)
        + b_ref[...], 0.0)


def _fc2(z656, fc2_W, fc2_b2d):
    kdim = ENC + 2 * TEMB
    return pl.pallas_call(
        _fc2_tc,
        grid=(NN * DEC // _MB2,),
        in_specs=[
            pl.BlockSpec((B, kdim), lambda m: (0, 0)),
            pl.BlockSpec((_MB2, kdim), lambda m: (m, 0)),
            pl.BlockSpec((1, _MB2), lambda m: (0, m)),
        ],
        out_specs=pl.BlockSpec((B, _MB2), lambda m: (0, m)),
        out_shape=jax.ShapeDtypeStruct((B, NN * DEC), jnp.float32),
    )(z656, fc2_W, fc2_b2d)


def _decpre_tc(zd_ref, waT_ref, wbT_ref, b_ref, a_ref, bo_ref):
    zd = zd_ref[...]
    a_ref[...] = jnp.dot(zd, waT_ref[...], preferred_element_type=jnp.float32)
    bo_ref[...] = (jnp.dot(zd, wbT_ref[...], preferred_element_type=jnp.float32)
                   + b_ref[...])


def _decpre(zdn, waT, wbT, b1_2d):
    return pl.pallas_call(
        _decpre_tc,
        grid=(N // _RB,),
        in_specs=[
            pl.BlockSpec((_RB, DEC), lambda i: (i, 0)),
            pl.BlockSpec((DEC, H1), lambda i: (0, 0)),
            pl.BlockSpec((DEC, H1), lambda i: (0, 0)),
            pl.BlockSpec((1, H1), lambda i: (0, 0)),
        ],
        out_specs=[
            pl.BlockSpec((_RB, H1), lambda i: (i, 0)),
            pl.BlockSpec((_RB, H1), lambda i: (i, 0)),
        ],
        out_shape=[
            jax.ShapeDtypeStruct((N, H1), jnp.float32),
            jax.ShapeDtypeStruct((N, H1), jnp.float32),
        ],
    )(zdn, waT, wbT, b1_2d)


def kernel(x, edge_index, edge_attr, hour, week,
           W1_rel, b1_rel, W1_root, W2_rel, b2_rel, W2_root,
           hour_table, week_table, fc_W, fc_b, fc2_W, fc2_b,
           dec_W1, dec_b1, dec_W2, dec_b2):
    src = edge_index[0]
    dst = edge_index[1]
    zeros = jnp.zeros((ACC_ROWS, F), jnp.float32)

    agg1 = _segsum(x, src, dst, edge_attr, zeros)
    h1 = _lin1(agg1, x, W1_rel.T, W1_root.T, b1_rel[None, :])

    y2, r2 = _lin2pre(h1, W2_rel.T, W2_root.T, b2_rel[None, :])
    agg2 = _segsum(y2, src, dst, edge_attr, zeros)

    emb_h, emb_w = _emb(hour[:, None], week[:, None], hour_table, week_table)
    z = _fcz(agg2.reshape(B, NN * H2), r2.reshape(B, NN * H2), fc_W,
             fc_W[:, NN * H2:NN * H2 + TEMB].T,
             fc_W[:, NN * H2 + TEMB:].T, fc_b[None, :], emb_h, emb_w)
    z656 = jnp.concatenate([z, emb_h, emb_w], axis=1)
    zd = _fc2(z656, fc2_W, fc2_b[None, :])

    zdn = zd.reshape(N, DEC)
    A, Bmb = _decpre(zdn, dec_W1[:, :DEC].T, dec_W1[:, DEC:].T, dec_b1[None, :])
    w2ext = jnp.concatenate(
        [dec_W2[0], jnp.full((LANES,), dec_b2[0], jnp.float32)])
    pred = _edgedec(A, Bmb, src, dst, w2ext)
    return pred[:, None]
